# Initial kernel scaffold; baseline (speedup 1.0000x reference)
import functools

import jax
import jax.numpy as jnp
from jax import lax
from jax.experimental import pallas as pl
from jax.experimental.pallas import tpu as pltpu
from jax.experimental.pallas import tpu_sc as plsc

NC = 2
NS = 16
LANES = 16
CHUNK = 128


def _sc_degree(dstp, n_acc, e_pad):
    ept = e_pad // (NC * NS)
    chunks = ept // CHUNK
    rpt = n_acc // NS
    mesh = plsc.VectorSubcoreMesh(core_axis_name="c", subcore_axis_name="s")

    zeros16 = jnp.zeros((n_acc, LANES), jnp.float32)
    ones16 = jnp.ones((CHUNK, LANES), jnp.float32)

    @functools.partial(
        pl.kernel,
        out_type=jax.ShapeDtypeStruct((NC * n_acc, LANES), jnp.float32),
        mesh=mesh,
        scratch_types=[
            pltpu.VMEM((CHUNK,), jnp.int32),
            pltpu.VMEM((CHUNK, LANES), jnp.float32),
            pltpu.VMEM_SHARED((n_acc, LANES), jnp.float32),
        ],
    )
    def deg_kernel(dst_hbm, z_hbm, ones_hbm, out_hbm, idx_v, ones_v, acc_sh):
        cid = lax.axis_index("c")
        sid = lax.axis_index("s")
        pltpu.sync_copy(ones_hbm, ones_v)
        pltpu.sync_copy(z_hbm.at[pl.ds(sid * rpt, rpt)],
                        acc_sh.at[pl.ds(sid * rpt, rpt)])
        plsc.subcore_barrier()
        tile_base = (cid * NS + sid) * ept

        def body(i, carry):
            pltpu.sync_copy(dst_hbm.at[pl.ds(tile_base + i * CHUNK, CHUNK)], idx_v)
            pltpu.sync_copy(ones_v, acc_sh.at[idx_v], add=True)
            return carry

        lax.fori_loop(0, chunks, body, 0)
        plsc.subcore_barrier()
        pltpu.sync_copy(acc_sh.at[pl.ds(sid * rpt, rpt)],
                        out_hbm.at[pl.ds(cid * n_acc + sid * rpt, rpt)])

    return deg_kernel(dstp, zeros16, ones16)


def _sc_message(g, srcp, dstp, n_acc, e_pad):
    d = g.shape[1]
    ept = e_pad // (NC * NS)
    chunks = ept // CHUNK
    rpt = n_acc // NS
    mesh = plsc.VectorSubcoreMesh(core_axis_name="c", subcore_axis_name="s")

    zeros = jnp.zeros((n_acc, d), jnp.float32)

    @functools.partial(
        pl.kernel,
        out_type=jax.ShapeDtypeStruct((NC * n_acc, d), jnp.float32),
        mesh=mesh,
        scratch_types=[
            pltpu.VMEM((CHUNK,), jnp.int32),
            pltpu.VMEM((CHUNK,), jnp.int32),
            pltpu.VMEM((CHUNK, d), jnp.float32),
            pltpu.VMEM_SHARED((n_acc, d), jnp.float32),
            pltpu.SemaphoreType.DMA,
        ],
    )
    def msg_kernel(g_hbm, src_hbm, dst_hbm, z_hbm, out_hbm,
                   sidx_v, didx_v, rows_v, acc_sh, sem):
        cid = lax.axis_index("c")
        sid = lax.axis_index("s")
        pltpu.sync_copy(z_hbm.at[pl.ds(sid * rpt, rpt)],
                        acc_sh.at[pl.ds(sid * rpt, rpt)])
        plsc.subcore_barrier()
        tile_base = (cid * NS + sid) * ept

        def body(i, carry):
            base = tile_base + i * CHUNK
            pltpu.sync_copy(src_hbm.at[pl.ds(base, CHUNK)], sidx_v)
            pltpu.sync_copy(dst_hbm.at[pl.ds(base, CHUNK)], didx_v)
            pltpu.async_copy(g_hbm.at[sidx_v], rows_v, sem).wait()
            pltpu.sync_copy(rows_v, acc_sh.at[didx_v], add=True)
            return carry

        lax.fori_loop(0, chunks, body, 0)
        plsc.subcore_barrier()
        pltpu.sync_copy(acc_sh.at[pl.ds(sid * rpt, rpt)],
                        out_hbm.at[pl.ds(cid * n_acc + sid * rpt, rpt)])

    return msg_kernel(g, srcp, dstp, zeros)


def _dinv_from_parts(degp_ref, n):
    deg = degp_ref[0] + degp_ref[1]
    return lax.rsqrt(deg[:n, 0:1] + 1.0)


def _tc_in(x, w1, degp, n):
    def body(x_ref, w_ref, degp_ref, g_ref):
        dinv = _dinv_from_parts(degp_ref, n)
        h = jnp.dot(x_ref[...], w_ref[...], preferred_element_type=jnp.float32)
        g_ref[...] = h * dinv

    return pl.pallas_call(
        body, out_shape=jax.ShapeDtypeStruct((n, x.shape[1]), jnp.float32),
    )(x, w1, degp)


def _tc_mid(parts, gprev, degp, b, w, n):
    def body(p_ref, g_ref, degp_ref, b_ref, w_ref, o_ref):
        dinv = _dinv_from_parts(degp_ref, n)
        s = p_ref[0, :n, :] + p_ref[1, :n, :] + g_ref[...]
        h = jnp.maximum(dinv * s + b_ref[...], 0.0)
        o_ref[...] = dinv * jnp.dot(h, w_ref[...],
                                    preferred_element_type=jnp.float32)

    return pl.pallas_call(
        body, out_shape=jax.ShapeDtypeStruct((n, w.shape[1]), jnp.float32),
    )(parts, gprev, degp, b, w)


def _tc_out(parts, gprev, degp, b, w, bfc, n):
    def body(p_ref, g_ref, degp_ref, b_ref, w_ref, bfc_ref, o_ref):
        dinv = _dinv_from_parts(degp_ref, n)
        s = p_ref[0, :n, :] + p_ref[1, :n, :] + g_ref[...]
        h = jnp.maximum(dinv * s + b_ref[...], 0.0)
        o_ref[...] = jnp.dot(h, w_ref[...],
                             preferred_element_type=jnp.float32) + bfc_ref[...]

    return pl.pallas_call(
        body, out_shape=jax.ShapeDtypeStruct((n, w.shape[1]), jnp.float32),
    )(parts, gprev, degp, b, w, bfc)


def kernel(x, edge_index, W1, b1, W2, b2, Wfc, bfc):
    n, d_in = x.shape
    e = edge_index.shape[1]

    n_acc = -(-(n + 1) // NS) * NS
    grain = NC * NS * CHUNK
    e_pad = -(-e // grain) * grain

    src = edge_index[0].astype(jnp.int32)
    dst = edge_index[1].astype(jnp.int32)
    srcp = jnp.concatenate([src, jnp.zeros((e_pad - e,), jnp.int32)])
    dstp = jnp.concatenate([dst, jnp.full((e_pad - e,), n, jnp.int32)])

    degp = _sc_degree(dstp, n_acc, e_pad).reshape(NC, n_acc, LANES)

    g1 = _tc_in(x, W1, degp, n)
    p1 = _sc_message(g1, srcp, dstp, n_acc, e_pad).reshape(NC, n_acc, d_in)
    g2 = _tc_mid(p1, g1, degp, b1.reshape(1, -1), W2, n)
    p2 = _sc_message(g2, srcp, dstp, n_acc, e_pad).reshape(NC, n_acc, d_in)
    out = _tc_out(p2, g2, degp, b2.reshape(1, -1), Wfc, bfc.reshape(1, -1), n)
    return out


# speedup vs baseline: 10.8357x; 10.8357x over previous
import functools

import jax
import jax.numpy as jnp
from jax import lax
from jax.experimental import pallas as pl
from jax.experimental.pallas import tpu as pltpu
from jax.experimental.pallas import tpu_sc as plsc

NC = 2
NS = 16
LANES = 16
CHUNK = 128


def _sc_degree(dstp, n_acc, e_pad):
    ept = e_pad // (NC * NS)
    chunks = ept // CHUNK
    rpt = n_acc // NS
    mesh = plsc.VectorSubcoreMesh(core_axis_name="c", subcore_axis_name="s")

    zeros16 = jnp.zeros((n_acc, LANES), jnp.float32)
    ones16 = jnp.ones((CHUNK, LANES), jnp.float32)

    @functools.partial(
        pl.kernel,
        out_type=jax.ShapeDtypeStruct((NC * n_acc, LANES), jnp.float32),
        mesh=mesh,
        scratch_types=[
            pltpu.VMEM((CHUNK,), jnp.int32),
            pltpu.VMEM((CHUNK, LANES), jnp.float32),
            pltpu.VMEM_SHARED((n_acc, LANES), jnp.float32),
        ],
        compiler_params=pltpu.CompilerParams(use_tc_tiling_on_sc=False),
    )
    def deg_kernel(dst_hbm, z_hbm, ones_hbm, out_hbm, idx_v, ones_v, acc_sh):
        cid = lax.axis_index("c")
        sid = lax.axis_index("s")
        pltpu.sync_copy(ones_hbm, ones_v)
        pltpu.sync_copy(z_hbm.at[pl.ds(sid * rpt, rpt)],
                        acc_sh.at[pl.ds(sid * rpt, rpt)])
        plsc.subcore_barrier()
        tile_base = (cid * NS + sid) * ept

        def body(i, carry):
            pltpu.sync_copy(dst_hbm.at[pl.ds(tile_base + i * CHUNK, CHUNK)], idx_v)
            pltpu.sync_copy(ones_v, acc_sh.at[idx_v], add=True)
            return carry

        lax.fori_loop(0, chunks, body, 0)
        plsc.subcore_barrier()
        pltpu.sync_copy(acc_sh.at[pl.ds(sid * rpt, rpt)],
                        out_hbm.at[pl.ds(cid * n_acc + sid * rpt, rpt)])

    return deg_kernel(dstp, zeros16, ones16)


def _sc_message(g, srcp, dstp, n_acc, e_pad):
    d = g.shape[1]
    ept = e_pad // (NC * NS)
    chunks = ept // CHUNK
    rpt = n_acc // NS
    mesh = plsc.VectorSubcoreMesh(core_axis_name="c", subcore_axis_name="s")

    zeros = jnp.zeros((n_acc, d), jnp.float32)

    @functools.partial(
        pl.kernel,
        out_type=jax.ShapeDtypeStruct((NC * n_acc, d), jnp.float32),
        mesh=mesh,
        scratch_types=[
            pltpu.VMEM((CHUNK,), jnp.int32),
            pltpu.VMEM((CHUNK,), jnp.int32),
            pltpu.VMEM((CHUNK, d), jnp.float32),
            pltpu.VMEM_SHARED((n_acc, d), jnp.float32),
            pltpu.SemaphoreType.DMA,
        ],
    )
    def msg_kernel(g_hbm, src_hbm, dst_hbm, z_hbm, out_hbm,
                   sidx_v, didx_v, rows_v, acc_sh, sem):
        cid = lax.axis_index("c")
        sid = lax.axis_index("s")
        pltpu.sync_copy(z_hbm.at[pl.ds(sid * rpt, rpt)],
                        acc_sh.at[pl.ds(sid * rpt, rpt)])
        plsc.subcore_barrier()
        tile_base = (cid * NS + sid) * ept

        def body(i, carry):
            base = tile_base + i * CHUNK
            pltpu.sync_copy(src_hbm.at[pl.ds(base, CHUNK)], sidx_v)
            pltpu.sync_copy(dst_hbm.at[pl.ds(base, CHUNK)], didx_v)
            pltpu.async_copy(g_hbm.at[sidx_v], rows_v, sem).wait()
            pltpu.sync_copy(rows_v, acc_sh.at[didx_v], add=True)
            return carry

        lax.fori_loop(0, chunks, body, 0)
        plsc.subcore_barrier()
        pltpu.sync_copy(acc_sh.at[pl.ds(sid * rpt, rpt)],
                        out_hbm.at[pl.ds(cid * n_acc + sid * rpt, rpt)])

    return msg_kernel(g, srcp, dstp, zeros)


def _dinv_from_parts(degp_ref, n):
    deg = degp_ref[0] + degp_ref[1]
    return lax.rsqrt(deg[:n, 0:1] + 1.0)


def _tc_in(x, w1, degp, n):
    def body(x_ref, w_ref, degp_ref, g_ref):
        dinv = _dinv_from_parts(degp_ref, n)
        h = jnp.dot(x_ref[...], w_ref[...], preferred_element_type=jnp.float32)
        g_ref[...] = h * dinv

    return pl.pallas_call(
        body, out_shape=jax.ShapeDtypeStruct((n, x.shape[1]), jnp.float32),
    )(x, w1, degp)


def _tc_mid(parts, gprev, degp, b, w, n):
    def body(p_ref, g_ref, degp_ref, b_ref, w_ref, o_ref):
        dinv = _dinv_from_parts(degp_ref, n)
        s = p_ref[0, :n, :] + p_ref[1, :n, :] + g_ref[...]
        h = jnp.maximum(dinv * s + b_ref[...], 0.0)
        o_ref[...] = dinv * jnp.dot(h, w_ref[...],
                                    preferred_element_type=jnp.float32)

    return pl.pallas_call(
        body, out_shape=jax.ShapeDtypeStruct((n, w.shape[1]), jnp.float32),
    )(parts, gprev, degp, b, w)


def _tc_out(parts, gprev, degp, b, w, bfc, n):
    def body(p_ref, g_ref, degp_ref, b_ref, w_ref, bfc_ref, o_ref):
        dinv = _dinv_from_parts(degp_ref, n)
        s = p_ref[0, :n, :] + p_ref[1, :n, :] + g_ref[...]
        h = jnp.maximum(dinv * s + b_ref[...], 0.0)
        o_ref[...] = jnp.dot(h, w_ref[...],
                             preferred_element_type=jnp.float32) + bfc_ref[...]

    return pl.pallas_call(
        body, out_shape=jax.ShapeDtypeStruct((n, w.shape[1]), jnp.float32),
    )(parts, gprev, degp, b, w, bfc)


def kernel(x, edge_index, W1, b1, W2, b2, Wfc, bfc):
    n, d_in = x.shape
    e = edge_index.shape[1]

    n_acc = -(-(n + 1) // (NS * 8)) * (NS * 8)
    grain = NC * NS * CHUNK
    e_pad = -(-e // grain) * grain

    src = edge_index[0].astype(jnp.int32)
    dst = edge_index[1].astype(jnp.int32)
    srcp = jnp.concatenate([src, jnp.zeros((e_pad - e,), jnp.int32)])
    dstp = jnp.concatenate([dst, jnp.full((e_pad - e,), n, jnp.int32)])

    degp = _sc_degree(dstp, n_acc, e_pad).reshape(NC, n_acc, LANES)

    g1 = _tc_in(x, W1, degp, n)
    p1 = _sc_message(g1, srcp, dstp, n_acc, e_pad).reshape(NC, n_acc, d_in)
    g2 = _tc_mid(p1, g1, degp, b1.reshape(1, -1), W2, n)
    p2 = _sc_message(g2, srcp, dstp, n_acc, e_pad).reshape(NC, n_acc, d_in)
    out = _tc_out(p2, g2, degp, b2.reshape(1, -1), Wfc, bfc.reshape(1, -1), n)
    return out
